# baseline (speedup 1.0000x reference)
import functools

import jax
import jax.numpy as jnp
from jax import lax
from jax.experimental import pallas as pl
from jax.experimental.pallas import tpu as pltpu
from jax.experimental.pallas import tpu_sc as plsc

_I = 8
_C = 16
_H = 256
_W = 256
_N = 64 * 64 * 64
_PLANE = _H * _W
_SENTINEL = _PLANE

_BN = 8192

_NC = 2
_NS = 16
_NW = _NC * _NS
_CH = 8192

_BF = 8192
_NBF = _N // _BF


def _bf(a):
    return a.astype(jnp.bfloat16).astype(jnp.float32)


def _project_uvw(trb, x, y, z):
    def col(a, j):
        return a[:, j:j + 1]

    u_num = col(trb, 0) * x + col(trb, 1) * y + col(trb, 2) * z + col(trb, 3)
    v_num = col(trb, 4) * x + col(trb, 5) * y + col(trb, 6) * z + col(trb, 7)
    w_num = col(trb, 8) * x + col(trb, 9) * y + col(trb, 10) * z + col(trb, 11)
    return u_num, v_num, w_num


def _uvw_to_valid(u_num, v_num, w_num):
    z_safe = jnp.where(jnp.abs(w_num) < 1e-8, 1e-8, w_num)
    u = u_num / z_safe
    v = v_num / z_safe
    ui = jnp.round(u).astype(jnp.int32)
    vi = jnp.round(v).astype(jnp.int32)
    valid = (ui >= 0) & (ui < _W) & (vi >= 0) & (vi < _H) & (w_num > 1e-8)
    return ui, vi, valid


def _tc_project_body(tr_ref, coords_ref, idx_ref):
    x = coords_ref[0:1, :]
    y = coords_ref[1:2, :]
    z = coords_ref[2:3, :]
    trb = _bf(tr_ref[...])
    u_num, v_num, w_num = _project_uvw(trb, _bf(x), _bf(y), _bf(z))
    ui, vi, valid = _uvw_to_valid(u_num, v_num, w_num)
    uc = jnp.clip(ui, 0, _W - 1)
    vc = jnp.clip(vi, 0, _H - 1)
    flat = vc * _W + uc
    idx_ref[...] = jnp.where(valid, flat, _SENTINEL)


def _tc_project(tr, coords):
    return pl.pallas_call(
        _tc_project_body,
        grid=(_N // _BN,),
        in_specs=[
            pl.BlockSpec((_I, 12), lambda n: (0, 0)),
            pl.BlockSpec((3, _BN), lambda n: (0, n)),
        ],
        out_specs=pl.BlockSpec((_I, _BN), lambda n: (0, n)),
        out_shape=jax.ShapeDtypeStruct((_I, _N), jnp.int32),
    )(tr, coords)


def _sc_gather_body(img_hbm, idx_hbm, out_hbm, plane_v,
                    idx_v0, idx_v1, val_v0, val_v1,
                    sem_a0, sem_a1, sem_b0, sem_b1):
    wid = lax.axis_index("s") * _NC + lax.axis_index("c")
    nchunks = _N // _CH
    bufs = ((idx_v0, val_v0, sem_a0, sem_b0), (idx_v1, val_v1, sem_a1,
                                               sem_b1))

    for r in range(4):
        plane = wid * 4 + r
        img = plane // _C
        ch = lax.rem(plane, _C)
        idx0 = img * _N
        out0 = (img * 21 + ch) * _N
        pltpu.sync_copy(img_hbm.at[pl.ds(plane * _PLANE, _PLANE)],
                        plane_v.at[pl.ds(0, _PLANE)])
        plane_v[pl.ds(_PLANE, 16)] = jnp.zeros((16,), jnp.float32)
        pltpu.async_copy(idx_hbm.at[pl.ds(idx0, _CH)], idx_v0, sem_a0)
        pltpu.async_copy(idx_hbm.at[pl.ds(idx0 + _CH, _CH)], idx_v1, sem_a1)

        def pair_body(kk, _, idx0=idx0, out0=out0):
            for b, (ibuf, vbuf, isem, osem) in enumerate(bufs):
                k = kk * 2 + b
                base = k * _CH
                pltpu.make_async_copy(idx_hbm.at[pl.ds(idx0 + base, _CH)],
                                      ibuf, isem).wait()

                @pl.when(kk > 0)
                def _(vbuf=vbuf, osem=osem, out0=out0, base=base):
                    pltpu.make_async_copy(
                        vbuf, out_hbm.at[pl.ds(out0 + base, _CH)],
                        osem).wait()

                @plsc.parallel_loop(0, _CH // 16, unroll=8)
                def _(j, ibuf=ibuf, vbuf=vbuf):
                    vi = ibuf[pl.ds(j * 16, 16)]
                    vbuf[pl.ds(j * 16, 16)] = plsc.load_gather(plane_v, [vi])

                @pl.when(k + 2 < nchunks)
                def _(ibuf=ibuf, isem=isem, idx0=idx0, base=base):
                    pltpu.async_copy(
                        idx_hbm.at[pl.ds(idx0 + base + 2 * _CH, _CH)], ibuf,
                        isem)

                pltpu.async_copy(vbuf, out_hbm.at[pl.ds(out0 + base, _CH)],
                                 osem)
            return 0

        lax.fori_loop(0, nchunks // 2, pair_body, 0)
        for b, (ibuf, vbuf, isem, osem) in enumerate(bufs):
            pltpu.make_async_copy(vbuf, out_hbm.at[pl.ds(out0, _CH)],
                                  osem).wait()


def _sc_gather(img1d, idx1d):
    mesh = plsc.VectorSubcoreMesh(core_axis_name="c", subcore_axis_name="s")
    return pl.kernel(
        _sc_gather_body,
        out_type=jax.ShapeDtypeStruct((_I * 21 * _N,), jnp.float32),
        mesh=mesh,
        compiler_params=pltpu.CompilerParams(needs_layout_passes=False),
        scratch_types=[
            pltpu.VMEM((_PLANE + 128,), jnp.float32),
            pltpu.VMEM((_CH,), jnp.int32),
            pltpu.VMEM((_CH,), jnp.int32),
            pltpu.VMEM((_CH,), jnp.float32),
            pltpu.VMEM((_CH,), jnp.float32),
            pltpu.SemaphoreType.DMA,
            pltpu.SemaphoreType.DMA,
            pltpu.SemaphoreType.DMA,
            pltpu.SemaphoreType.DMA,
        ],
    )(img1d, idx1d)


def _tc_dense_body(gin_ref, tr_ref, tcw_ref, coords_ref, out_ref):
    del gin_ref
    i = pl.program_id(0)
    d = pl.program_id(1)
    x = coords_ref[0:1, :]
    y = coords_ref[1:2, :]
    z = coords_ref[2:3, :]
    trb = _bf(tr_ref[pl.ds(i, 1), :])
    tc = tcw_ref[pl.ds(i, 1), :]
    tcb = _bf(tc)
    u_num, v_num, w_num = _project_uvw(trb, _bf(x), _bf(y), _bf(z))
    _, _, valid = _uvw_to_valid(u_num, v_num, w_num)

    def col(a, j):
        return a[:, j:j + 1]

    xb = _bf(x)
    yb = _bf(y)
    zb = _bf(z)
    depth = (col(tcb, 8) * xb + col(tcb, 9) * yb + col(tcb, 10) * zb
             + col(tcb, 11))
    t0 = col(tc, 3)
    t1 = col(tc, 7)
    t2 = col(tc, 11)
    cc0 = -(col(tc, 0) * t0 + col(tc, 4) * t1 + col(tc, 8) * t2)
    cc1 = -(col(tc, 1) * t0 + col(tc, 5) * t1 + col(tc, 9) * t2)
    cc2 = -(col(tc, 2) * t0 + col(tc, 6) * t1 + col(tc, 10) * t2)
    dx = x - cc0
    dy = y - cc1
    dz = z - cc2
    nrm = jnp.maximum(jnp.sqrt(dx * dx + dy * dy + dz * dz), 1e-8)
    validf = valid.astype(jnp.float32)

    val = depth
    val = jnp.where(d == 1, validf, val)
    val = jnp.where(d == 2, dx / nrm, val)
    val = jnp.where(d == 3, dy / nrm, val)
    val = jnp.where(d == 4, dz / nrm, val)
    out_ref[...] = val.reshape(_BF)


def _tc_dense(g, tr, tcw, coords):
    return pl.pallas_call(
        _tc_dense_body,
        grid=(_I, 5, _NBF),
        in_specs=[
            pl.BlockSpec(memory_space=pl.ANY),
            pl.BlockSpec((_I, 12), lambda i, d, n: (0, 0)),
            pl.BlockSpec((_I, 16), lambda i, d, n: (0, 0)),
            pl.BlockSpec((3, _BF), lambda i, d, n: (0, n)),
        ],
        out_specs=pl.BlockSpec(
            (_BF,), lambda i, d, n: ((i * 21 + _C + d) * _NBF + n,)),
        out_shape=jax.ShapeDtypeStruct((_I * 21 * _N,), jnp.float32),
        input_output_aliases={0: 0},
    )(g, tr, tcw, coords)


def kernel(images, transformations, T_cw, coordinates):
    B, I, C, H, W = images.shape
    _, _, Xd, Zd, Yd = coordinates.shape
    coords = coordinates.reshape(3, _N)
    tr = transformations.reshape(I, 12)
    tcw = T_cw.reshape(I, 16)
    idx = _tc_project(tr, coords)
    img1d = images.reshape(I * C * H * W)
    g = _sc_gather(img1d, idx.reshape(-1))
    full = _tc_dense(g, tr, tcw, coords)
    input_grid = full.reshape(B, I, 21, Xd, Zd, Yd)
    return (input_grid, coordinates)


# speedup vs baseline: 1.7234x; 1.7234x over previous
import functools

import jax
import jax.numpy as jnp
from jax import lax
from jax.experimental import pallas as pl
from jax.experimental.pallas import tpu as pltpu
from jax.experimental.pallas import tpu_sc as plsc

_I = 8
_C = 16
_H = 256
_W = 256
_N = 64 * 64 * 64
_PLANE = _H * _W
_SENTINEL = _PLANE

_BN = 8192

_NC = 2
_NS = 16
_NW = _NC * _NS
_CH = 8192

_BF = 8192
_NBF = _N // _BF


def _bf(a):
    return a.astype(jnp.bfloat16).astype(jnp.float32)


def _project_uvw(trb, x, y, z):
    def col(a, j):
        return a[:, j:j + 1]

    u_num = col(trb, 0) * x + col(trb, 1) * y + col(trb, 2) * z + col(trb, 3)
    v_num = col(trb, 4) * x + col(trb, 5) * y + col(trb, 6) * z + col(trb, 7)
    w_num = col(trb, 8) * x + col(trb, 9) * y + col(trb, 10) * z + col(trb, 11)
    return u_num, v_num, w_num


def _uvw_to_valid(u_num, v_num, w_num):
    z_safe = jnp.where(jnp.abs(w_num) < 1e-8, 1e-8, w_num)
    u = u_num / z_safe
    v = v_num / z_safe
    ui = jnp.round(u).astype(jnp.int32)
    vi = jnp.round(v).astype(jnp.int32)
    valid = (ui >= 0) & (ui < _W) & (vi >= 0) & (vi < _H) & (w_num > 1e-8)
    return ui, vi, valid


def _tc_project_body(tr_ref, coords_ref, idx_ref):
    x = coords_ref[0:1, :]
    y = coords_ref[1:2, :]
    z = coords_ref[2:3, :]
    trb = _bf(tr_ref[...])
    u_num, v_num, w_num = _project_uvw(trb, _bf(x), _bf(y), _bf(z))
    ui, vi, valid = _uvw_to_valid(u_num, v_num, w_num)
    uc = jnp.clip(ui, 0, _W - 1)
    vc = jnp.clip(vi, 0, _H - 1)
    flat = vc * _W + uc
    idx_ref[...] = jnp.where(valid, flat, _SENTINEL)


def _tc_project(tr, coords):
    return pl.pallas_call(
        _tc_project_body,
        grid=(_N // _BN,),
        in_specs=[
            pl.BlockSpec((_I, 12), lambda n: (0, 0)),
            pl.BlockSpec((3, _BN), lambda n: (0, n)),
        ],
        out_specs=pl.BlockSpec((_I, _BN), lambda n: (0, n)),
        out_shape=jax.ShapeDtypeStruct((_I, _N), jnp.int32),
    )(tr, coords)


def _sc_gather_body(img_hbm, idx_hbm, out_hbm, plane_v,
                    idx_v0, idx_v1, val_v0, val_v1,
                    sem_a0, sem_a1, sem_b0, sem_b1):
    wid = lax.axis_index("s") * _NC + lax.axis_index("c")
    nchunks = _N // _CH
    bufs = ((idx_v0, val_v0, sem_a0, sem_b0), (idx_v1, val_v1, sem_a1,
                                               sem_b1))

    for r in range(4):
        plane = wid * 4 + r
        img = plane // _C
        ch = lax.rem(plane, _C)
        idx0 = img * _N
        out0 = (img * 21 + ch) * _N
        pltpu.sync_copy(img_hbm.at[pl.ds(plane * _PLANE, _PLANE)],
                        plane_v.at[pl.ds(0, _PLANE)])
        plane_v[pl.ds(_PLANE, 16)] = jnp.zeros((16,), jnp.float32)
        pltpu.async_copy(idx_hbm.at[pl.ds(idx0, _CH)], idx_v0, sem_a0)
        pltpu.async_copy(idx_hbm.at[pl.ds(idx0 + _CH, _CH)], idx_v1, sem_a1)

        def pair_body(kk, _, idx0=idx0, out0=out0):
            for b, (ibuf, vbuf, isem, osem) in enumerate(bufs):
                k = kk * 2 + b
                base = k * _CH
                pltpu.make_async_copy(idx_hbm.at[pl.ds(idx0 + base, _CH)],
                                      ibuf, isem).wait()

                @pl.when(kk > 0)
                def _(vbuf=vbuf, osem=osem, out0=out0, base=base):
                    pltpu.make_async_copy(
                        vbuf, out_hbm.at[pl.ds(out0 + base, _CH)],
                        osem).wait()

                @plsc.parallel_loop(0, _CH // 16, unroll=8)
                def _(j, ibuf=ibuf, vbuf=vbuf):
                    vi = ibuf[pl.ds(j * 16, 16)]
                    vbuf[pl.ds(j * 16, 16)] = plsc.load_gather(plane_v, [vi])

                @pl.when(k + 2 < nchunks)
                def _(ibuf=ibuf, isem=isem, idx0=idx0, base=base):
                    pltpu.async_copy(
                        idx_hbm.at[pl.ds(idx0 + base + 2 * _CH, _CH)], ibuf,
                        isem)

                pltpu.async_copy(vbuf, out_hbm.at[pl.ds(out0 + base, _CH)],
                                 osem)
            return 0

        lax.fori_loop(0, nchunks // 2, pair_body, 0)
        for b, (ibuf, vbuf, isem, osem) in enumerate(bufs):
            pltpu.make_async_copy(vbuf, out_hbm.at[pl.ds(out0, _CH)],
                                  osem).wait()


def _sc_gather(img1d, idx1d):
    mesh = plsc.VectorSubcoreMesh(core_axis_name="c", subcore_axis_name="s")
    return pl.kernel(
        _sc_gather_body,
        out_type=jax.ShapeDtypeStruct((_I * 21 * _N,), jnp.float32),
        mesh=mesh,
        compiler_params=pltpu.CompilerParams(needs_layout_passes=False),
        scratch_types=[
            pltpu.VMEM((_PLANE + 128,), jnp.float32),
            pltpu.VMEM((_CH,), jnp.int32),
            pltpu.VMEM((_CH,), jnp.int32),
            pltpu.VMEM((_CH,), jnp.float32),
            pltpu.VMEM((_CH,), jnp.float32),
            pltpu.SemaphoreType.DMA,
            pltpu.SemaphoreType.DMA,
            pltpu.SemaphoreType.DMA,
            pltpu.SemaphoreType.DMA,
        ],
    )(img1d, idx1d)


_BD = _N // 2


def _tc_dense_body(gin_ref, tr_ref, tcw_ref, coords_ref, out_ref, cache_ref):
    del gin_ref
    i = pl.program_id(0)
    d = pl.program_id(2)

    @pl.when(d == 0)
    def _():
        x = coords_ref[0:1, :]
        y = coords_ref[1:2, :]
        z = coords_ref[2:3, :]
        trb = _bf(tr_ref[pl.ds(i, 1), :])
        tc = tcw_ref[pl.ds(i, 1), :]
        tcb = _bf(tc)
        u_num, v_num, w_num = _project_uvw(trb, _bf(x), _bf(y), _bf(z))
        _, _, valid = _uvw_to_valid(u_num, v_num, w_num)

        def col(a, j):
            return a[:, j:j + 1]

        xb = _bf(x)
        yb = _bf(y)
        zb = _bf(z)
        depth = (col(tcb, 8) * xb + col(tcb, 9) * yb + col(tcb, 10) * zb
                 + col(tcb, 11))
        t0 = col(tc, 3)
        t1 = col(tc, 7)
        t2 = col(tc, 11)
        cc0 = -(col(tc, 0) * t0 + col(tc, 4) * t1 + col(tc, 8) * t2)
        cc1 = -(col(tc, 1) * t0 + col(tc, 5) * t1 + col(tc, 9) * t2)
        cc2 = -(col(tc, 2) * t0 + col(tc, 6) * t1 + col(tc, 10) * t2)
        dx = x - cc0
        dy = y - cc1
        dz = z - cc2
        nrm = jnp.maximum(jnp.sqrt(dx * dx + dy * dy + dz * dz), 1e-8)
        validf = valid.astype(jnp.float32)
        cache_ref[...] = jnp.concatenate(
            [depth, validf, dx / nrm, dy / nrm, dz / nrm], axis=0)

    out_ref[...] = cache_ref[pl.ds(d, 1), :].reshape(_BD)


def _tc_dense(g, tr, tcw, coords):
    return pl.pallas_call(
        _tc_dense_body,
        grid=(_I, 2, 5),
        in_specs=[
            pl.BlockSpec(memory_space=pl.ANY),
            pl.BlockSpec((_I, 12), lambda i, h, d: (0, 0)),
            pl.BlockSpec((_I, 16), lambda i, h, d: (0, 0)),
            pl.BlockSpec((3, _BD), lambda i, h, d: (0, h)),
        ],
        out_specs=pl.BlockSpec(
            (_BD,), lambda i, h, d: ((i * 21 + _C + d) * 2 + h,)),
        out_shape=jax.ShapeDtypeStruct((_I * 21 * _N,), jnp.float32),
        input_output_aliases={0: 0},
        scratch_shapes=[pltpu.VMEM((5, _BD), jnp.float32)],
    )(g, tr, tcw, coords)


def kernel(images, transformations, T_cw, coordinates):
    B, I, C, H, W = images.shape
    _, _, Xd, Zd, Yd = coordinates.shape
    coords = coordinates.reshape(3, _N)
    tr = transformations.reshape(I, 12)
    tcw = T_cw.reshape(I, 16)
    idx = _tc_project(tr, coords)
    img1d = images.reshape(I * C * H * W)
    g = _sc_gather(img1d, idx.reshape(-1))
    full = _tc_dense(g, tr, tcw, coords)
    input_grid = full.reshape(B, I, 21, Xd, Zd, Yd)
    return (input_grid, coordinates)


# speedup vs baseline: 2.1352x; 1.2390x over previous
import functools

import jax
import jax.numpy as jnp
from jax import lax
from jax.experimental import pallas as pl
from jax.experimental.pallas import tpu as pltpu
from jax.experimental.pallas import tpu_sc as plsc

_I = 8
_C = 16
_H = 256
_W = 256
_N = 64 * 64 * 64
_PLANE = _H * _W
_SENTINEL = _PLANE

_BN = 8192

_NC = 2
_NS = 16
_NW = _NC * _NS
_CH = 8192


def _bf(a):
    return a.astype(jnp.bfloat16).astype(jnp.float32)


def _tc_project_body(tr_ref, tcw_ref, coords_ref, idx_ref, dense_ref):
    x = coords_ref[0:1, :]
    y = coords_ref[1:2, :]
    z = coords_ref[2:3, :]
    xb = _bf(x)
    yb = _bf(y)
    zb = _bf(z)
    tr = tr_ref[...]
    tc = tcw_ref[...]
    trb = _bf(tr)
    tcb = _bf(tc)

    def col(a, j):
        return a[:, j:j + 1]

    u_num = col(trb, 0) * xb + col(trb, 1) * yb + col(trb, 2) * zb + col(trb, 3)
    v_num = col(trb, 4) * xb + col(trb, 5) * yb + col(trb, 6) * zb + col(trb, 7)
    w_num = col(trb, 8) * xb + col(trb, 9) * yb + col(trb, 10) * zb + col(trb, 11)

    z_safe = jnp.where(jnp.abs(w_num) < 1e-8, 1e-8, w_num)
    u = u_num / z_safe
    v = v_num / z_safe
    ui = jnp.round(u).astype(jnp.int32)
    vi = jnp.round(v).astype(jnp.int32)
    valid = (ui >= 0) & (ui < _W) & (vi >= 0) & (vi < _H) & (w_num > 1e-8)
    uc = jnp.clip(ui, 0, _W - 1)
    vc = jnp.clip(vi, 0, _H - 1)
    flat = vc * _W + uc
    idx_ref[...] = jnp.where(valid, flat, _SENTINEL)

    depth = (col(tcb, 8) * xb + col(tcb, 9) * yb + col(tcb, 10) * zb
             + col(tcb, 11))

    t0 = col(tc, 3)
    t1 = col(tc, 7)
    t2 = col(tc, 11)
    cc0 = -(col(tc, 0) * t0 + col(tc, 4) * t1 + col(tc, 8) * t2)
    cc1 = -(col(tc, 1) * t0 + col(tc, 5) * t1 + col(tc, 9) * t2)
    cc2 = -(col(tc, 2) * t0 + col(tc, 6) * t1 + col(tc, 10) * t2)
    dx = x - cc0
    dy = y - cc1
    dz = z - cc2
    nrm = jnp.maximum(jnp.sqrt(dx * dx + dy * dy + dz * dz), 1e-8)
    validf = valid.astype(jnp.float32)
    dense_ref[...] = jnp.stack(
        [depth, validf, dx / nrm, dy / nrm, dz / nrm], axis=0)


def _tc_project(tr, tcw, coords):
    grid = (_N // _BN,)
    return pl.pallas_call(
        _tc_project_body,
        grid=grid,
        in_specs=[
            pl.BlockSpec((_I, 12), lambda n: (0, 0)),
            pl.BlockSpec((_I, 16), lambda n: (0, 0)),
            pl.BlockSpec((3, _BN), lambda n: (0, n)),
        ],
        out_specs=[
            pl.BlockSpec((_I, _BN), lambda n: (0, n)),
            pl.BlockSpec((5, _I, _BN), lambda n: (0, 0, n)),
        ],
        out_shape=[
            jax.ShapeDtypeStruct((_I, _N), jnp.int32),
            jax.ShapeDtypeStruct((5, _I, _N), jnp.float32),
        ],
    )(tr, tcw, coords)


def _sc_gather_body(img_hbm, idx_hbm, dense_hbm, out_hbm, plane_v,
                    idx_v0, idx_v1, val_v0, val_v1, den_v0, den_v1,
                    sem_a0, sem_a1, sem_b0, sem_b1):
    wid = lax.axis_index("s") * _NC + lax.axis_index("c")
    nchunks = _N // _CH
    bufs = ((idx_v0, val_v0, sem_a0, sem_b0), (idx_v1, val_v1, sem_a1,
                                               sem_b1))

    for r in range(4):
        plane = wid * 4 + r
        img = plane // _C
        ch = lax.rem(plane, _C)
        idx0 = img * _N
        out0 = (img * 21 + ch) * _N
        with jax.named_scope("plane_load"):
            pltpu.sync_copy(img_hbm.at[pl.ds(plane * _PLANE, _PLANE)],
                            plane_v.at[pl.ds(0, _PLANE)])
        plane_v[pl.ds(_PLANE, 16)] = jnp.zeros((16,), jnp.float32)
        pltpu.async_copy(idx_hbm.at[pl.ds(idx0, _CH)], idx_v0, sem_a0)
        pltpu.async_copy(idx_hbm.at[pl.ds(idx0 + _CH, _CH)], idx_v1, sem_a1)

        def pair_body(kk, _, idx0=idx0, out0=out0):
            for b, (ibuf, vbuf, isem, osem) in enumerate(bufs):
                k = kk * 2 + b
                base = k * _CH
                pltpu.make_async_copy(idx_hbm.at[pl.ds(idx0 + base, _CH)],
                                      ibuf, isem).wait()

                @pl.when(kk > 0)
                def _(vbuf=vbuf, osem=osem, out0=out0, base=base):
                    pltpu.make_async_copy(
                        vbuf, out_hbm.at[pl.ds(out0 + base, _CH)],
                        osem).wait()

                @plsc.parallel_loop(0, _CH // 16, unroll=8)
                def _(j, ibuf=ibuf, vbuf=vbuf):
                    vi = ibuf[pl.ds(j * 16, 16)]
                    vbuf[pl.ds(j * 16, 16)] = plsc.load_gather(plane_v, [vi])

                @pl.when(k + 2 < nchunks)
                def _(ibuf=ibuf, isem=isem, idx0=idx0, base=base):
                    pltpu.async_copy(
                        idx_hbm.at[pl.ds(idx0 + base + 2 * _CH, _CH)], ibuf,
                        isem)

                pltpu.async_copy(vbuf, out_hbm.at[pl.ds(out0 + base, _CH)],
                                 osem)
            return 0

        with jax.named_scope("gather_phase"):
            lax.fori_loop(0, nchunks // 2, pair_body, 0)
            for b, (ibuf, vbuf, isem, osem) in enumerate(bufs):
                pltpu.make_async_copy(vbuf, out_hbm.at[pl.ds(out0, _CH)],
                                      osem).wait()

    dbufs = ((den_v0, sem_a0, sem_b0), (den_v1, sem_a1, sem_b1))
    for rep in range(2):
        task = wid + _NW * rep

        @pl.when(task < 40)
        def _(task=task):
            d = task // _I
            img = lax.rem(task, _I)
            src0 = (d * _I + img) * _N
            dst0 = (img * 21 + _C + d) * _N

            def pair_body(kk, _):
                for b, (vbuf, isem, osem) in enumerate(dbufs):
                    k = kk * 2 + b
                    base = k * _CH

                    @pl.when(kk > 0)
                    def _(vbuf=vbuf, osem=osem, base=base):
                        pltpu.make_async_copy(
                            vbuf, out_hbm.at[pl.ds(dst0 + base, _CH)],
                            osem).wait()

                    pltpu.async_copy(dense_hbm.at[pl.ds(src0 + base, _CH)],
                                     vbuf, isem)
                    pltpu.make_async_copy(
                        dense_hbm.at[pl.ds(src0 + base, _CH)], vbuf,
                        isem).wait()
                    pltpu.async_copy(vbuf, out_hbm.at[pl.ds(dst0 + base,
                                                            _CH)], osem)
                return 0

            with jax.named_scope("dense_phase"):
                lax.fori_loop(0, nchunks // 2, pair_body, 0)
                for b, (vbuf, isem, osem) in enumerate(dbufs):
                    pltpu.make_async_copy(vbuf, out_hbm.at[pl.ds(dst0, _CH)],
                                          osem).wait()


def _sc_gather(img2d, idx, dense):
    mesh = plsc.VectorSubcoreMesh(core_axis_name="c", subcore_axis_name="s")
    return pl.kernel(
        _sc_gather_body,
        out_type=jax.ShapeDtypeStruct((_I * 21 * _N,), jnp.float32),
        mesh=mesh,
        compiler_params=pltpu.CompilerParams(needs_layout_passes=False),
        scratch_types=[
            pltpu.VMEM((_PLANE + 128,), jnp.float32),
            pltpu.VMEM((_CH,), jnp.int32),
            pltpu.VMEM((_CH,), jnp.int32),
            pltpu.VMEM((_CH,), jnp.float32),
            pltpu.VMEM((_CH,), jnp.float32),
            pltpu.VMEM((_CH,), jnp.float32),
            pltpu.VMEM((_CH,), jnp.float32),
            pltpu.SemaphoreType.DMA,
            pltpu.SemaphoreType.DMA,
            pltpu.SemaphoreType.DMA,
            pltpu.SemaphoreType.DMA,
        ],
    )(img2d, idx, dense)


def kernel(images, transformations, T_cw, coordinates):
    B, I, C, H, W = images.shape
    _, _, Xd, Zd, Yd = coordinates.shape
    coords = coordinates.reshape(3, _N)
    tr = transformations.reshape(I, 12)
    tcw = T_cw.reshape(I, 16)
    idx, dense = _tc_project(tr, tcw, coords)
    img1d = images.reshape(I * C * H * W)
    out = _sc_gather(img1d, idx.reshape(-1), dense.reshape(-1))
    input_grid = out.reshape(B, I, 21, Xd, Zd, Yd)
    return (input_grid, coordinates)
